# baseline (speedup 1.0000x reference)
import functools

import jax
import jax.numpy as jnp
from jax import lax
from jax.experimental import pallas as pl
from jax.experimental.pallas import tpu as pltpu
from jax.experimental.pallas import tpu_sc as plsc

HASH = 5001
D = 64
DP = 128
NF = 6
B = 16384
DOUT = 256

NC, NS, L = 2, 16, 16
NW = NC * NS
BPW = B // NW
RCH = 64
NCH = BPW // RCH

PGRP = 16
PSUB = 4
PCH = 64


def _sc_gather_body(rel_id, fork_num, block_num, relfilenode, rel_kind, position,
                    p_rel_id, p_fork_num, p_block_num, p_relfilenode, p_rel_kind,
                    t_position,
                    o0, o1, o2, o3, o4, o5,
                    idx_v, buf0, buf1, pos_vidx, tb, posbuf,
                    sem0, sem1, psem):
    cid = lax.axis_index("c")
    sid = lax.axis_index("s")
    wid = sid * NC + cid
    base = wid * BPW

    idx_refs = (rel_id, fork_num, block_num, relfilenode, rel_kind)
    tabs = (p_rel_id, p_fork_num, p_block_num, p_relfilenode, p_rel_kind)
    outs = (o0, o1, o2, o3, o4, o5)

    def stage_chunk(j, c):
        for f in range(5):
            pltpu.sync_copy(idx_refs[f].at[pl.ds(base + j * RCH, RCH)],
                            idx_v.at[f, j])
        for f in range(4):
            for i in range(RCH // L):
                v = idx_v[f, j, pl.ds(i * L, L)]
                idx_v[f, j, pl.ds(i * L, L)] = lax.rem(v, jnp.int32(HASH))
        return c

    lax.fori_loop(0, NCH, stage_chunk, 0)

    pltpu.sync_copy(position.at[pl.ds(base, BPW)], pos_vidx)

    for f in range(5):
        def gather_pair(j2, c, f=f):
            h0 = pltpu.async_copy(tabs[f].at[idx_v.at[f, 2 * j2]], buf0, sem0)
            h1 = pltpu.async_copy(tabs[f].at[idx_v.at[f, 2 * j2 + 1]], buf1,
                                  sem1)
            h0.wait()
            pltpu.sync_copy(buf0,
                            outs[f].at[pl.ds(base + (2 * j2) * RCH, RCH)])
            h1.wait()
            pltpu.sync_copy(buf1,
                            outs[f].at[pl.ds(base + (2 * j2 + 1) * RCH, RCH)])
            return c

        lax.fori_loop(0, NCH // 2, gather_pair, 0)

    def pos_chunk(jc, c):
        def pos_group(g, c2):
            i0 = jc * PCH + g * PGRP
            gv = pos_vidx[pl.ds(i0, PGRP)]
            for q in range(PGRP // PSUB):
                handles = []
                for s in range(PSUB):
                    idx = gv[q * PSUB + s]
                    tile = pl.multiple_of((idx // 8) * 8, 8)
                    handles.append(pltpu.async_copy(
                        t_position.at[pl.ds(tile, 8)], tb.at[s], psem))
                for s in range(PSUB):
                    handles[s].wait()
                    sub = gv[q * PSUB + s] % 8
                    for k in range(D // L):
                        posbuf[g * PGRP + q * PSUB + s, pl.ds(k * L, L)] = \
                            tb[s, sub, pl.ds(k * L, L)]
            return c2

        lax.fori_loop(0, PCH // PGRP, pos_group, 0)
        pltpu.sync_copy(posbuf, o5.at[pl.ds(base + jc * PCH, PCH)])
        return c

    lax.fori_loop(0, BPW // PCH, pos_chunk, 0)


_sc_gather = functools.partial(
    pl.kernel,
    out_type=tuple(jax.ShapeDtypeStruct((B, DP), jnp.float32)
                   for _ in range(5))
    + (jax.ShapeDtypeStruct((B, D), jnp.float32),),
    mesh=plsc.VectorSubcoreMesh(core_axis_name="c", subcore_axis_name="s"),
    scratch_types=[
        pltpu.VMEM((5, NCH, RCH), jnp.int32),
        pltpu.VMEM((RCH, DP), jnp.float32),
        pltpu.VMEM((RCH, DP), jnp.float32),
        pltpu.VMEM((BPW,), jnp.int32),
        pltpu.VMEM((PSUB, 8, D), jnp.float32),
        pltpu.VMEM((PCH, D), jnp.float32),
        pltpu.SemaphoreType.DMA,
        pltpu.SemaphoreType.DMA,
        pltpu.SemaphoreType.DMA,
    ],
)(_sc_gather_body)


BM = 1024


def _dense_body(x0, x1, x2, x3, x4, x5, w_ref, b_ref, o_ref):
    xs = (x0, x1, x2, x3, x4)
    acc = b_ref[...].astype(jnp.float32)
    for f in range(5):
        acc = acc + jnp.dot(xs[f][...], w_ref[f],
                            preferred_element_type=jnp.float32)
    acc = acc + jnp.dot(x5[...], w_ref[5, :D],
                        preferred_element_type=jnp.float32)
    o_ref[...] = jnp.maximum(acc, 0.0)


def _dense(xs, w3d, b2d):
    return pl.pallas_call(
        _dense_body,
        grid=(B // BM,),
        in_specs=[pl.BlockSpec((BM, DP), lambda i: (i, 0)) for _ in range(5)]
        + [
            pl.BlockSpec((BM, D), lambda i: (i, 0)),
            pl.BlockSpec((NF, DP, DOUT), lambda i: (0, 0, 0)),
            pl.BlockSpec((1, DOUT), lambda i: (0, 0)),
        ],
        out_specs=pl.BlockSpec((BM, DOUT), lambda i: (i, 0)),
        out_shape=jax.ShapeDtypeStruct((B, DOUT), jnp.float32),
    )(*xs, w3d, b2d)


def kernel(rel_id, fork_num, block_num, relfilenode, rel_kind, position,
           t_rel_id, t_fork_num, t_block_num, t_relfilenode, t_rel_kind,
           t_position, W, b):
    pad = lambda t: jnp.pad(t, ((0, 0), (0, DP - D)))
    xs = _sc_gather(rel_id, fork_num, block_num, relfilenode, rel_kind,
                    position, pad(t_rel_id), pad(t_fork_num),
                    pad(t_block_num), pad(t_relfilenode), pad(t_rel_kind),
                    t_position)
    w3d = jnp.pad(W.reshape(NF, D, DOUT), ((0, 0), (0, DP - D), (0, 0)))
    return _dense(xs, w3d, b.reshape(1, DOUT))


# speedup vs baseline: 1.1978x; 1.1978x over previous
import functools

import jax
import jax.numpy as jnp
from jax import lax
from jax.experimental import pallas as pl
from jax.experimental.pallas import tpu as pltpu
from jax.experimental.pallas import tpu_sc as plsc

HASH = 5001
D = 64
DP = 128
NF = 6
B = 16384
DOUT = 256

NC, NS, L = 2, 16, 16
NW = NC * NS
BPW = B // NW
RCH = 64
NCH = BPW // RCH
NBUF = 4

PGRP = 16
PCH = 64


def _k1_body(rel_id, fork_num, block_num, relfilenode, rel_kind,
             p_rel_id, p_fork_num, p_block_num, p_relfilenode, p_rel_kind,
             o0, o1, o2, o3, o4,
             idx_v, buf0, buf1, buf2, buf3, sem0, sem1, sem2, sem3):
    cid = lax.axis_index("c")
    sid = lax.axis_index("s")
    wid = sid * NC + cid
    base = wid * BPW

    idx_refs = (rel_id, fork_num, block_num, relfilenode, rel_kind)
    tabs = (p_rel_id, p_fork_num, p_block_num, p_relfilenode, p_rel_kind)
    outs = (o0, o1, o2, o3, o4)
    bufs = (buf0, buf1, buf2, buf3)
    sems = (sem0, sem1, sem2, sem3)

    def stage_chunk(j, c):
        for f in range(5):
            pltpu.sync_copy(idx_refs[f].at[pl.ds(base + j * RCH, RCH)],
                            idx_v.at[f, j])
        for f in range(4):
            for i in range(RCH // L):
                v = idx_v[f, j, pl.ds(i * L, L)]
                idx_v[f, j, pl.ds(i * L, L)] = lax.rem(v, jnp.int32(HASH))
        return c

    lax.fori_loop(0, NCH, stage_chunk, 0)

    units = [(f, j) for f in range(5) for j in range(NCH)]

    def gfire(u, b):
        f, j = units[u]
        return pltpu.async_copy(tabs[f].at[idx_v.at[f, j]], bufs[b],
                                sems[b])

    def wdesc(u, b):
        f, j = units[u]
        return pltpu.make_async_copy(
            bufs[b], outs[f].at[pl.ds(base + j * RCH, RCH)], sems[b])

    pend_g = {u: gfire(u, u % NBUF) for u in range(NBUF)}
    pend_w = {}
    for u in range(len(units)):
        b = u % NBUF
        pend_g.pop(u).wait()
        pend_w[u] = wdesc(u, b)
        pend_w[u].start()
        if u + NBUF < len(units):
            pend_w.pop(u).wait()
            pend_g[u + NBUF] = gfire(u + NBUF, b)
    for u in list(pend_w):
        pend_w.pop(u).wait()


_k1 = functools.partial(
    pl.kernel,
    out_type=tuple(jax.ShapeDtypeStruct((B, DP), jnp.float32)
                   for _ in range(5)),
    mesh=plsc.VectorSubcoreMesh(core_axis_name="c", subcore_axis_name="s"),
    scratch_types=[pltpu.VMEM((5, NCH, RCH), jnp.int32)]
    + [pltpu.VMEM((RCH, DP), jnp.float32) for _ in range(NBUF)]
    + [pltpu.SemaphoreType.DMA for _ in range(NBUF)],
)(_k1_body)


def _k2_body(position, t_position, o5,
             pos_vidx, tb0, tb1, posbuf, psem0, psem1):
    cid = lax.axis_index("c")
    sid = lax.axis_index("s")
    wid = sid * NC + cid
    base = wid * BPW

    pltpu.sync_copy(position.at[pl.ds(base, BPW)], pos_vidx)

    tbs = (tb0, tb1)
    psems = (psem0, psem1)

    def fire_group(g, tb, psem):
        gv = pos_vidx[pl.ds(g * PGRP, PGRP)]
        for s in range(PGRP):
            idx = gv[s]
            tile = pl.multiple_of((idx // 8) * 8, 8)
            pltpu.async_copy(t_position.at[pl.ds(tile, 8)], tb.at[s], psem)
        return gv

    def drain_group(g, gv, tb, psem, row0):
        for s in range(PGRP):
            pltpu.make_async_copy(t_position.at[pl.ds(0, 8)], tb.at[s],
                                  psem).wait()
            sub = gv[s] % 8
            for k in range(D // L):
                posbuf[row0 + s, pl.ds(k * L, L)] = tb[s, sub, pl.ds(k * L, L)]

    def pos_chunk(jc, c):
        g0 = jc * (PCH // PGRP)
        gv_prev = fire_group(g0, tbs[0], psems[0])
        for q in range(PCH // PGRP):
            parity = q % 2
            nxt = 1 - parity
            if q + 1 < PCH // PGRP:
                gv_next = fire_group(g0 + q + 1, tbs[nxt], psems[nxt])
            drain_group(g0 + q, gv_prev, tbs[parity], psems[parity],
                        q * PGRP)
            if q + 1 < PCH // PGRP:
                gv_prev = gv_next
        pltpu.sync_copy(posbuf, o5.at[pl.ds(base + jc * PCH, PCH)])
        return c

    lax.fori_loop(0, BPW // PCH, pos_chunk, 0)


_k2 = functools.partial(
    pl.kernel,
    out_type=jax.ShapeDtypeStruct((B, D), jnp.float32),
    mesh=plsc.VectorSubcoreMesh(core_axis_name="c", subcore_axis_name="s"),
    scratch_types=[
        pltpu.VMEM((BPW,), jnp.int32),
        pltpu.VMEM((PGRP, 8, D), jnp.float32),
        pltpu.VMEM((PGRP, 8, D), jnp.float32),
        pltpu.VMEM((PCH, D), jnp.float32),
        pltpu.SemaphoreType.DMA,
        pltpu.SemaphoreType.DMA,
    ],
)(_k2_body)


BM = 1024


def _dense_body(x0, x1, x2, x3, x4, x5, w_ref, b_ref, o_ref):
    xs = (x0, x1, x2, x3, x4)
    acc = b_ref[...].astype(jnp.float32)
    for f in range(5):
        acc = acc + jnp.dot(xs[f][...], w_ref[f],
                            preferred_element_type=jnp.float32)
    acc = acc + jnp.dot(x5[...], w_ref[5, :D],
                        preferred_element_type=jnp.float32)
    o_ref[...] = jnp.maximum(acc, 0.0)


def _dense(xs, w3d, b2d):
    return pl.pallas_call(
        _dense_body,
        grid=(B // BM,),
        in_specs=[pl.BlockSpec((BM, DP), lambda i: (i, 0)) for _ in range(5)]
        + [
            pl.BlockSpec((BM, D), lambda i: (i, 0)),
            pl.BlockSpec((NF, DP, DOUT), lambda i: (0, 0, 0)),
            pl.BlockSpec((1, DOUT), lambda i: (0, 0)),
        ],
        out_specs=pl.BlockSpec((BM, DOUT), lambda i: (i, 0)),
        out_shape=jax.ShapeDtypeStruct((B, DOUT), jnp.float32),
    )(*xs, w3d, b2d)


def kernel(rel_id, fork_num, block_num, relfilenode, rel_kind, position,
           t_rel_id, t_fork_num, t_block_num, t_relfilenode, t_rel_kind,
           t_position, W, b):
    pad = lambda t: jnp.pad(t, ((0, 0), (0, DP - D)))
    xs = _k1(rel_id, fork_num, block_num, relfilenode, rel_kind,
             pad(t_rel_id), pad(t_fork_num), pad(t_block_num),
             pad(t_relfilenode), pad(t_rel_kind))
    x5 = _k2(position, t_position)
    w3d = jnp.pad(W.reshape(NF, D, DOUT), ((0, 0), (0, DP - D), (0, 0)))
    return _dense(xs + (x5,), w3d, b.reshape(1, DOUT))
